# Initial kernel scaffold; baseline (speedup 1.0000x reference)
import jax
import jax.numpy as jnp
from jax.experimental import pallas as pl


def kernel(query_address, addresses):
    raise NotImplementedError("write your pallas kernel here")


# speedup vs baseline: 1.8380x; 1.8380x over previous
import jax
import jax.numpy as jnp
from jax.experimental import pallas as pl
from jax.experimental.pallas import tpu as pltpu

_TEMPERATURE = 0.07
_EPS = 1e-8


def _dsdm_kernel(q_ref, a_ref, o_ref, acc_ref, l_ref, qs_ref):
    j = pl.program_id(1)
    nj = pl.num_programs(1)

    @pl.when(j == 0)
    def _init():
        q = q_ref[...]
        qn = jnp.sqrt(jnp.sum(q * q, axis=1, keepdims=True))
        qs = q * (1.0 / (jnp.maximum(qn, _EPS) * _TEMPERATURE))
        qs_ref[...] = qs.astype(jnp.bfloat16)
        acc_ref[...] = jnp.zeros_like(acc_ref)
        l_ref[...] = jnp.zeros_like(l_ref)

    a = a_ref[...]
    an = jnp.sqrt(jnp.sum(a * a, axis=1, keepdims=True))
    ahat = (a * (1.0 / jnp.maximum(an, _EPS))).astype(jnp.bfloat16)
    s = jax.lax.dot_general(
        qs_ref[...], ahat, (((1,), (1,)), ((), ())),
        preferred_element_type=jnp.float32,
    )
    p = jnp.exp(s)
    l_ref[...] += jnp.sum(p, axis=1, keepdims=True)
    acc_ref[...] += jax.lax.dot_general(
        p.astype(jnp.bfloat16), a.astype(jnp.bfloat16),
        (((1,), (0,)), ((), ())),
        preferred_element_type=jnp.float32,
    )

    @pl.when(j == nj - 1)
    def _done():
        o_ref[...] = acc_ref[...] / l_ref[...]


def kernel(query_address, addresses):
    Q, D = query_address.shape
    N, _ = addresses.shape
    BQ = min(512, Q)
    BN = min(2048, N)

    return pl.pallas_call(
        _dsdm_kernel,
        grid=(Q // BQ, N // BN),
        in_specs=[
            pl.BlockSpec((BQ, D), lambda i, j: (i, 0)),
            pl.BlockSpec((BN, D), lambda i, j: (j, 0)),
        ],
        out_specs=pl.BlockSpec((BQ, D), lambda i, j: (i, 0)),
        out_shape=jax.ShapeDtypeStruct((Q, D), jnp.float32),
        scratch_shapes=[
            pltpu.VMEM((BQ, D), jnp.float32),
            pltpu.VMEM((BQ, 1), jnp.float32),
            pltpu.VMEM((BQ, D), jnp.bfloat16),
        ],
        compiler_params=pltpu.CompilerParams(
            dimension_semantics=("parallel", "arbitrary"),
        ),
    )(query_address, addresses)


# speedup vs baseline: 2.1017x; 1.1434x over previous
import math

import jax
import jax.numpy as jnp
from jax.experimental import pallas as pl
from jax.experimental.pallas import tpu as pltpu

_TEMPERATURE = 0.07
_EPS = 1e-8
_QSCALE = 1.0 / (_TEMPERATURE * math.log(2.0))


def _dsdm_kernel(q_ref, a_ref, o_ref, acc_ref, l_ref, qs_ref):
    j = pl.program_id(0)
    nj = pl.num_programs(0)

    @pl.when(j == 0)
    def _init():
        q = q_ref[...]
        qn = jnp.sqrt(jnp.sum(q * q, axis=1, keepdims=True))
        qs = q * (_QSCALE / jnp.maximum(qn, _EPS))
        qs_ref[...] = qs.astype(jnp.bfloat16)
        acc_ref[...] = jnp.zeros_like(acc_ref)
        l_ref[...] = jnp.zeros_like(l_ref)

    a = a_ref[...]
    an = jnp.sqrt(jnp.sum(a * a, axis=1, keepdims=True))
    ainv = (1.0 / jnp.maximum(an, _EPS)).astype(jnp.bfloat16)
    abf = a.astype(jnp.bfloat16)
    ahat = abf * ainv
    s = jax.lax.dot_general(
        qs_ref[...], ahat, (((1,), (1,)), ((), ())),
        preferred_element_type=jnp.float32,
    )
    p = jnp.exp2(s)
    l_ref[...] += jnp.sum(p, axis=1, keepdims=True)
    acc_ref[...] += jax.lax.dot_general(
        p.astype(jnp.bfloat16), abf, (((1,), (0,)), ((), ())),
        preferred_element_type=jnp.float32,
    )

    @pl.when(j == nj - 1)
    def _done():
        o_ref[...] = acc_ref[...] / l_ref[...]


def kernel(query_address, addresses):
    Q, D = query_address.shape
    N, _ = addresses.shape
    BN = min(2048, N)

    return pl.pallas_call(
        _dsdm_kernel,
        grid=(N // BN,),
        in_specs=[
            pl.BlockSpec((Q, D), lambda j: (0, 0)),
            pl.BlockSpec((BN, D), lambda j: (j, 0)),
        ],
        out_specs=pl.BlockSpec((Q, D), lambda j: (0, 0)),
        out_shape=jax.ShapeDtypeStruct((Q, D), jnp.float32),
        scratch_shapes=[
            pltpu.VMEM((Q, D), jnp.float32),
            pltpu.VMEM((Q, 1), jnp.float32),
            pltpu.VMEM((Q, D), jnp.bfloat16),
        ],
        compiler_params=pltpu.CompilerParams(
            dimension_semantics=("arbitrary",),
        ),
    )(query_address, addresses)
